# baseline (speedup 1.0000x reference)
import functools

import jax
import jax.numpy as jnp
from jax import lax
from jax.experimental import pallas as pl
from jax.experimental.pallas import tpu as pltpu
from jax.experimental.pallas import tpu_sc as plsc

VOCAB = 100000
D_EMB = 128
MAX_SEQ = 2048
D_MODEL = 1024
LN_EPS = 1e-12

BATCH = 4
SEQ = 2048
N_TOK = BATCH * SEQ
NW = 32
MAX_GATHER = 64

TC_BLOCK = 2048

CHUNKS = (8192,)

_INV_D = 1.0 / D_EMB


@functools.cache
def _make_sc_gather(tok_start, ntok):
  tok_per_tile = ntok // NW
  n_gath = -(-tok_per_tile // MAX_GATHER)
  g = tok_per_tile // n_gath
  assert g * n_gath == tok_per_tile and g <= MAX_GATHER
  mesh = plsc.VectorSubcoreMesh(core_axis_name="c", subcore_axis_name="s")

  @functools.partial(
      pl.kernel,
      mesh=mesh,
      out_type=jax.ShapeDtypeStruct((ntok, D_EMB), jnp.float32),
      scratch_types=[
          pltpu.VMEM((n_gath, g), jnp.int32),
          pltpu.VMEM((tok_per_tile, D_EMB), jnp.float32),
          pltpu.SemaphoreType.DMA,
          pltpu.SemaphoreType.DMA,
          pltpu.SemaphoreType.DMA,
      ],
  )
  def gather_kernel(ids_hbm, wtab_hbm, wout_hbm, idx_v, wrows_v, semi,
                    semg, semw):
    c = lax.axis_index("c")
    s = lax.axis_index("s")
    wid = s * 2 + c
    tok0 = tok_start + wid * tok_per_tile
    b = tok0 // SEQ
    off = tok0 % SEQ
    cpi = [
        pltpu.async_copy(ids_hbm.at[b, pl.ds(off + j * g, g)], idx_v.at[j],
                         semi)
        for j in range(n_gath)
    ]
    cpg = []
    for j in range(n_gath):
      cpi[j].wait()
      cpg.append(
          pltpu.async_copy(wtab_hbm.at[idx_v.at[j]],
                           wrows_v.at[pl.ds(j * g, g)], semg))
    cpw = []
    for j in range(n_gath):
      cpg[j].wait()
      cpw.append(
          pltpu.async_copy(wrows_v.at[pl.ds(j * g, g)],
                           wout_hbm.at[pl.ds(wid * tok_per_tile + j * g, g)],
                           semw))
    for cp in cpw:
      cp.wait()

  return gather_kernel


def _tc_body(gath_ref, tid_ref, tdiff_ref, posq_ref, dk_ref, db_ref, *rest):
  out_ref = rest[-1]
  t = tid_ref[...].astype(jnp.float32)
  x = gath_ref[...] + posq_ref[...] + t * tdiff_ref[...]
  s1 = jnp.sum(x, axis=1, keepdims=True)
  s2 = jnp.sum(x * x, axis=1, keepdims=True)
  mean = s1 * _INV_D
  var = s2 * _INV_D - mean * mean
  y = (x - mean) * lax.rsqrt(var + LN_EPS)
  out_ref[...] = (
      jnp.dot(y.astype(jnp.bfloat16), dk_ref[...],
              preferred_element_type=jnp.float32)
      + db_ref[...])


def _tc_call(tok_start, ntok, gathered, tids, tdiff, posq, dkp, dbp, buf):
  base = tok_start // TC_BLOCK
  steps = ntok // TC_BLOCK

  in_specs = [
      pl.BlockSpec((TC_BLOCK, D_EMB), lambda i: (i, 0)),
      pl.BlockSpec((TC_BLOCK, 1), lambda i: (base + i, 0)),
      pl.BlockSpec((1, D_EMB), lambda i: (0, 0)),
      pl.BlockSpec((TC_BLOCK, D_EMB), lambda i: (0, 0)),
      pl.BlockSpec((D_EMB, D_MODEL), lambda i: (0, 0)),
      pl.BlockSpec((1, D_MODEL), lambda i: (0, 0)),
  ]
  args = [gathered, tids, tdiff, posq, dkp, dbp]
  aliases = {}
  if buf is not None:
    in_specs.append(pl.BlockSpec(memory_space=pl.ANY))
    args.append(buf)
    aliases = {6: 0}

  return pl.pallas_call(
      _tc_body,
      grid=(steps,),
      in_specs=in_specs,
      out_specs=pl.BlockSpec((TC_BLOCK, D_MODEL), lambda i: (base + i, 0)),
      out_shape=jax.ShapeDtypeStruct((N_TOK, D_MODEL), jnp.float32),
      input_output_aliases=aliases,
  )(*args)


def kernel(input_ids, type_ids, word_emb, pos_emb, type_emb, ln_scale,
           ln_bias, dense_kernel, dense_bias):
  batch, seq = input_ids.shape

  ids = input_ids.astype(jnp.int32)
  tids = type_ids.reshape(batch * seq, 1).astype(jnp.int32)

  starts = []
  t = 0
  for n in CHUNKS:
    starts.append(t)
    t += n
  assert t == N_TOK

  gathered = [
      _make_sc_gather(s, n)(ids, word_emb)
      for s, n in zip(starts, CHUNKS)
  ]

  posq = pos_emb.reshape(MAX_SEQ, D_EMB)[:seq] + type_emb[0:1, :]
  tdiff = (type_emb[1:2, :] - type_emb[0:1, :])
  dkp = (ln_scale[:, None] * dense_kernel).astype(jnp.bfloat16)
  dbp = (ln_bias @ dense_kernel + dense_bias).reshape(1, D_MODEL)

  buf = None
  for s, n, g in zip(starts, CHUNKS, gathered):
    buf = _tc_call(s, n, g, tids, tdiff, posq, dkp, dbp, buf)

  return buf.reshape(batch, seq, D_MODEL)


# speedup vs baseline: 1.0045x; 1.0045x over previous
import functools

import jax
import jax.numpy as jnp
from jax import lax
from jax.experimental import pallas as pl
from jax.experimental.pallas import tpu as pltpu
from jax.experimental.pallas import tpu_sc as plsc

VOCAB = 100000
D_EMB = 128
MAX_SEQ = 2048
D_MODEL = 1024
LN_EPS = 1e-12

BATCH = 4
SEQ = 2048
N_TOK = BATCH * SEQ
NW = 32
MAX_GATHER = 128

TC_BLOCK = 2048

CHUNKS = (8192,)

_INV_D = 1.0 / D_EMB


@functools.cache
def _make_sc_gather(tok_start, ntok):
  tok_per_tile = ntok // NW
  n_gath = -(-tok_per_tile // MAX_GATHER)
  g = tok_per_tile // n_gath
  assert g * n_gath == tok_per_tile and g <= MAX_GATHER
  mesh = plsc.VectorSubcoreMesh(core_axis_name="c", subcore_axis_name="s")

  @functools.partial(
      pl.kernel,
      mesh=mesh,
      out_type=jax.ShapeDtypeStruct((ntok, D_EMB), jnp.float32),
      scratch_types=[
          pltpu.VMEM((n_gath, g), jnp.int32),
          pltpu.VMEM((tok_per_tile, D_EMB), jnp.float32),
          pltpu.SemaphoreType.DMA,
          pltpu.SemaphoreType.DMA,
          pltpu.SemaphoreType.DMA,
      ],
  )
  def gather_kernel(ids_hbm, wtab_hbm, wout_hbm, idx_v, wrows_v, semi,
                    semg, semw):
    c = lax.axis_index("c")
    s = lax.axis_index("s")
    wid = s * 2 + c
    tok0 = tok_start + wid * tok_per_tile
    b = tok0 // SEQ
    off = tok0 % SEQ
    cpi = [
        pltpu.async_copy(ids_hbm.at[b, pl.ds(off + j * g, g)], idx_v.at[j],
                         semi)
        for j in range(n_gath)
    ]
    cpg = []
    for j in range(n_gath):
      cpi[j].wait()
      cpg.append(
          pltpu.async_copy(wtab_hbm.at[idx_v.at[j]],
                           wrows_v.at[pl.ds(j * g, g)], semg))
    cpw = []
    for j in range(n_gath):
      cpg[j].wait()
      cpw.append(
          pltpu.async_copy(wrows_v.at[pl.ds(j * g, g)],
                           wout_hbm.at[pl.ds(wid * tok_per_tile + j * g, g)],
                           semw))
    for cp in cpw:
      cp.wait()

  return gather_kernel


def _tc_body(gath_ref, tid_ref, tdiff_ref, posq_ref, dk_ref, db_ref, *rest):
  out_ref = rest[-1]
  t = tid_ref[...].astype(jnp.float32)
  x = gath_ref[...] + posq_ref[...] + t * tdiff_ref[...]
  s1 = jnp.sum(x, axis=1, keepdims=True)
  s2 = jnp.sum(x * x, axis=1, keepdims=True)
  mean = s1 * _INV_D
  var = s2 * _INV_D - mean * mean
  y = (x - mean) * lax.rsqrt(var + LN_EPS)
  out_ref[...] = (
      jnp.dot(y.astype(jnp.bfloat16), dk_ref[...],
              preferred_element_type=jnp.float32)
      + db_ref[...])


def _tc_call(tok_start, ntok, gathered, tids, tdiff, posq, dkp, dbp, buf):
  base = tok_start // TC_BLOCK
  steps = ntok // TC_BLOCK

  in_specs = [
      pl.BlockSpec((TC_BLOCK, D_EMB), lambda i: (i, 0)),
      pl.BlockSpec((TC_BLOCK, 1), lambda i: (base + i, 0)),
      pl.BlockSpec((1, D_EMB), lambda i: (0, 0)),
      pl.BlockSpec((TC_BLOCK, D_EMB), lambda i: (0, 0)),
      pl.BlockSpec((D_EMB, D_MODEL), lambda i: (0, 0)),
      pl.BlockSpec((1, D_MODEL), lambda i: (0, 0)),
  ]
  args = [gathered, tids, tdiff, posq, dkp, dbp]
  aliases = {}
  if buf is not None:
    in_specs.append(pl.BlockSpec(memory_space=pl.ANY))
    args.append(buf)
    aliases = {6: 0}

  return pl.pallas_call(
      _tc_body,
      grid=(steps,),
      in_specs=in_specs,
      out_specs=pl.BlockSpec((TC_BLOCK, D_MODEL), lambda i: (base + i, 0)),
      out_shape=jax.ShapeDtypeStruct((N_TOK, D_MODEL), jnp.float32),
      input_output_aliases=aliases,
  )(*args)


def kernel(input_ids, type_ids, word_emb, pos_emb, type_emb, ln_scale,
           ln_bias, dense_kernel, dense_bias):
  batch, seq = input_ids.shape

  ids = input_ids.astype(jnp.int32)
  tids = type_ids.reshape(batch * seq, 1).astype(jnp.int32)

  starts = []
  t = 0
  for n in CHUNKS:
    starts.append(t)
    t += n
  assert t == N_TOK

  gathered = [
      _make_sc_gather(s, n)(ids, word_emb)
      for s, n in zip(starts, CHUNKS)
  ]

  posq = pos_emb.reshape(MAX_SEQ, D_EMB)[:seq] + type_emb[0:1, :]
  tdiff = (type_emb[1:2, :] - type_emb[0:1, :])
  dkp = (ln_scale[:, None] * dense_kernel).astype(jnp.bfloat16)
  dbp = (ln_bias @ dense_kernel + dense_bias).reshape(1, D_MODEL)

  buf = None
  for s, n, g in zip(starts, CHUNKS, gathered):
    buf = _tc_call(s, n, g, tids, tdiff, posq, dkp, dbp, buf)

  return buf.reshape(batch, seq, D_MODEL)
